# Initial kernel scaffold; baseline (speedup 1.0000x reference)
import jax
import jax.numpy as jnp
from jax.experimental import pallas as pl


def kernel(x, edge_index, edge_attribute, i, dummy, W_u, b_u, W_l1, b_l1, root, bias):
    raise NotImplementedError("write your pallas kernel here")


# speedup vs baseline: 8.0413x; 8.0413x over previous
import functools

import jax
import jax.numpy as jnp
from jax import lax
from jax.experimental import pallas as pl
from jax.experimental.pallas import tpu as pltpu
from jax.experimental.pallas import tpu_sc as plsc

_B = 128
_U = 200
_D = 16
_N = _B * _U
_E = 409600
_NC = 2
_NS = 16
_TILE_EDGES = _E // (_NC * _NS)
_CHUNK = 128
_NCHUNK = _TILE_EDGES // _CHUNK
_ROWS_PER_TILE = _N // _NS
_ZROWS = 100


def _node_body(xf_ref, wu_ref, bu_ref, wpq_ref, root_ref, bias_ref,
               t_ref, hroot_ref):
    h = jnp.maximum(xf_ref[...] * wu_ref[...] + bu_ref[...], 0.0)
    t_ref[...] = jnp.dot(h, wpq_ref[...], preferred_element_type=jnp.float32)
    hroot_ref[...] = (
        jnp.dot(h, root_ref[...], preferred_element_type=jnp.float32)
        + bias_ref[...])


def _node_phase(xf, wu, bu, wpq, root, bias):
    return pl.pallas_call(
        _node_body,
        out_shape=(
            jax.ShapeDtypeStruct((_N, 2 * _D), jnp.float32),
            jax.ShapeDtypeStruct((_N, _D), jnp.float32),
        ),
    )(xf, wu, bu, wpq, root, bias)


def _sc_body(t_hbm, src_hbm, dst_hbm, ea_hbm, out_hbm,
             sidx, didx, eav, rows, msg, zbuf, acc_sh, sem):
    cid = lax.axis_index("c")
    sid = lax.axis_index("s")

    zero16 = jnp.zeros((_D,), jnp.float32)

    def zfill(j, carry):
        zbuf[j, pl.ds(0, _D)] = zero16
        zbuf[j, pl.ds(_D, _D)] = zero16
        return carry

    lax.fori_loop(0, _ZROWS, zfill, 0)
    row0 = sid * _ROWS_PER_TILE
    for k in range(_ROWS_PER_TILE // _ZROWS):
        pltpu.sync_copy(zbuf, acc_sh.at[pl.ds(row0 + k * _ZROWS, _ZROWS)])

    one16 = jnp.ones((_D,), jnp.float32)

    def ofill(j, carry):
        msg[j, pl.ds(_D, _D)] = one16
        return carry

    lax.fori_loop(0, _CHUNK, ofill, 0)
    plsc.subcore_barrier()

    wid = cid * _NS + sid
    ebase = wid * _TILE_EDGES

    def chunk_body(ci, carry):
        off = ebase + ci * _CHUNK
        pltpu.sync_copy(src_hbm.at[pl.ds(off, _CHUNK)], sidx)
        pltpu.sync_copy(dst_hbm.at[pl.ds(off, _CHUNK)], didx)
        pltpu.sync_copy(ea_hbm.at[pl.ds(off, _CHUNK)], eav)
        pltpu.async_copy(t_hbm.at[sidx], rows, sem).wait()

        def group_body(g, c2):
            base = g * _D
            ev = eav[pl.ds(base, _D)]
            for k in range(_D):
                j = base + k
                p = rows[j, pl.ds(0, _D)]
                q = rows[j, pl.ds(_D, _D)]
                msg[j, pl.ds(0, _D)] = p * ev[k] + q
            return c2

        lax.fori_loop(0, _CHUNK // _D, group_body, 0)
        pltpu.sync_copy(msg, acc_sh.at[didx], add=True)
        return carry

    lax.fori_loop(0, _NCHUNK, chunk_body, 0)
    plsc.subcore_barrier()

    pltpu.sync_copy(acc_sh.at[pl.ds(row0, _ROWS_PER_TILE)],
                    out_hbm.at[cid, pl.ds(row0, _ROWS_PER_TILE)])


def _edge_phase(t, src, dst, ea):
    mesh = plsc.VectorSubcoreMesh(core_axis_name="c", subcore_axis_name="s")
    f = pl.kernel(
        _sc_body,
        mesh=mesh,
        compiler_params=pltpu.CompilerParams(use_tc_tiling_on_sc=False),
        out_type=jax.ShapeDtypeStruct((_NC, _N, 2 * _D), jnp.float32),
        scratch_types=[
            pltpu.VMEM((_CHUNK,), jnp.int32),
            pltpu.VMEM((_CHUNK,), jnp.int32),
            pltpu.VMEM((_CHUNK,), jnp.float32),
            pltpu.VMEM((_CHUNK, 2 * _D), jnp.float32),
            pltpu.VMEM((_CHUNK, 2 * _D), jnp.float32),
            pltpu.VMEM((_ZROWS, 2 * _D), jnp.float32),
            pltpu.VMEM_SHARED((_N, 2 * _D), jnp.float32),
            pltpu.SemaphoreType.DMA,
        ],
    )
    return f(t, src, dst, ea)


_CBLK = 3200


def _combine_body(acc_ref, hroot_ref, out_ref):
    s = acc_ref[0, :, 0, :] + acc_ref[1, :, 0, :]
    c = acc_ref[0, :, 1, :] + acc_ref[1, :, 1, :]
    out_ref[...] = (
        jnp.where(c > 0.0, s / jnp.maximum(c, 1.0), 0.0) + hroot_ref[...])


def _combine(acc, hroot):
    acc4 = acc.reshape(_NC, _N, 2, _D)
    return pl.pallas_call(
        _combine_body,
        grid=(_N // _CBLK,),
        in_specs=[
            pl.BlockSpec((_NC, _CBLK, 2, _D), lambda ib: (0, ib, 0, 0)),
            pl.BlockSpec((_CBLK, _D), lambda ib: (ib, 0)),
        ],
        out_specs=pl.BlockSpec((_CBLK, _D), lambda ib: (ib, 0)),
        out_shape=jax.ShapeDtypeStruct((_N, _D), jnp.float32),
    )(acc4, hroot)


def kernel(x, edge_index, edge_attribute, i, dummy,
           W_u, b_u, W_l1, b_l1, root, bias):
    xf = x.reshape(_N, 1)
    src = edge_index[0]
    dst = edge_index[1]
    ea = edge_attribute.reshape(_E)
    wpq = jnp.concatenate(
        [W_l1.reshape(_D, _D), b_l1.reshape(_D, _D)], axis=1)
    t, hroot = _node_phase(xf, W_u, b_u.reshape(1, _D), wpq,
                           root, bias.reshape(1, _D))
    acc = _edge_phase(t, src, dst, ea)
    return _combine(acc, hroot)


# speedup vs baseline: 13.7488x; 1.7098x over previous
import functools

import jax
import jax.numpy as jnp
from jax import lax
from jax.experimental import pallas as pl
from jax.experimental.pallas import tpu as pltpu
from jax.experimental.pallas import tpu_sc as plsc

_B = 128
_U = 200
_D = 16
_N = _B * _U
_E = 409600
_NC = 2
_NS = 16
_TILE_EDGES = _E // (_NC * _NS)
_CHUNK = 128
_NCHUNK = _TILE_EDGES // _CHUNK
_ROWS_PER_TILE = _N // _NS
_ZROWS = 100


def _node_body(xf_ref, wu_ref, bu_ref, wpq_ref, root_ref, bias_ref,
               t_ref, hroot_ref):
    h = jnp.maximum(xf_ref[...] * wu_ref[...] + bu_ref[...], 0.0)
    t_ref[...] = jnp.dot(h, wpq_ref[...], preferred_element_type=jnp.float32)
    hroot_ref[...] = (
        jnp.dot(h, root_ref[...], preferred_element_type=jnp.float32)
        + bias_ref[...])


def _node_phase(xf, wu, bu, wpq, root, bias):
    return pl.pallas_call(
        _node_body,
        out_shape=(
            jax.ShapeDtypeStruct((_N, 2 * _D), jnp.float32),
            jax.ShapeDtypeStruct((_N, _D), jnp.float32),
        ),
    )(xf, wu, bu, wpq, root, bias)


def _sc_body(t_hbm, src_hbm, dst_hbm, ea_hbm, out_hbm,
             sall, dall, eall, rows, msg, zbuf, acc_sh, gsem):
    cid = lax.axis_index("c")
    sid = lax.axis_index("s")
    wid = cid * _NS + sid

    pltpu.sync_copy(src_hbm.at[pl.ds(wid * _NCHUNK, _NCHUNK)], sall)
    pltpu.sync_copy(dst_hbm.at[pl.ds(wid * _NCHUNK, _NCHUNK)], dall)
    pltpu.sync_copy(ea_hbm.at[pl.ds(wid * _NCHUNK, _NCHUNK)], eall)

    zero16 = jnp.zeros((_D,), jnp.float32)

    def zfill(j, carry):
        zbuf[j, pl.ds(0, _D)] = zero16
        zbuf[j, pl.ds(_D, _D)] = zero16
        return carry

    lax.fori_loop(0, _ZROWS, zfill, 0)
    row0 = sid * _ROWS_PER_TILE
    for k in range(_ROWS_PER_TILE // _ZROWS):
        pltpu.sync_copy(zbuf, acc_sh.at[pl.ds(row0 + k * _ZROWS, _ZROWS)])

    one16 = jnp.ones((_D,), jnp.float32)

    def ofill(j, carry):
        msg[0, j, pl.ds(_D, _D)] = one16
        msg[1, j, pl.ds(_D, _D)] = one16
        return carry

    lax.fori_loop(0, _CHUNK, ofill, 0)
    plsc.subcore_barrier()

    pltpu.async_copy(t_hbm.at[sall.at[0]], rows.at[0], gsem)

    def do_chunk(ci, b):
        nci = ci + 1

        @pl.when(nci < _NCHUNK)
        def _():
            pltpu.async_copy(t_hbm.at[sall.at[nci]], rows.at[1 - b], gsem)

        pltpu.make_async_copy(t_hbm.at[sall.at[ci]], rows.at[b], gsem).wait()

        def group_body(g, c2):
            base = g * _D
            ev = eall[ci, pl.ds(base, _D)]
            for k in range(_D):
                j = base + k
                p = rows[b, j, pl.ds(0, _D)]
                q = rows[b, j, pl.ds(_D, _D)]
                msg[b, j, pl.ds(0, _D)] = p * ev[k] + q
            return c2

        lax.fori_loop(0, _CHUNK // _D, group_body, 0)
        pltpu.sync_copy(msg.at[b], acc_sh.at[dall.at[ci]], add=True)

    def pair_body(h, carry):
        do_chunk(h * 2, 0)
        do_chunk(h * 2 + 1, 1)
        return carry

    lax.fori_loop(0, _NCHUNK // 2, pair_body, 0)
    plsc.subcore_barrier()

    pltpu.sync_copy(acc_sh.at[pl.ds(row0, _ROWS_PER_TILE)],
                    out_hbm.at[cid, pl.ds(row0, _ROWS_PER_TILE)])


def _edge_phase(t, src, dst, ea):
    mesh = plsc.VectorSubcoreMesh(core_axis_name="c", subcore_axis_name="s")
    f = pl.kernel(
        _sc_body,
        mesh=mesh,
        compiler_params=pltpu.CompilerParams(use_tc_tiling_on_sc=False),
        out_type=jax.ShapeDtypeStruct((_NC, _N, 2 * _D), jnp.float32),
        scratch_types=[
            pltpu.VMEM((_NCHUNK, _CHUNK), jnp.int32),
            pltpu.VMEM((_NCHUNK, _CHUNK), jnp.int32),
            pltpu.VMEM((_NCHUNK, _CHUNK), jnp.float32),
            pltpu.VMEM((2, _CHUNK, 2 * _D), jnp.float32),
            pltpu.VMEM((2, _CHUNK, 2 * _D), jnp.float32),
            pltpu.VMEM((_ZROWS, 2 * _D), jnp.float32),
            pltpu.VMEM_SHARED((_N, 2 * _D), jnp.float32),
            pltpu.SemaphoreType.DMA,
        ],
    )
    src2 = src.reshape(_E // _CHUNK, _CHUNK)
    dst2 = dst.reshape(_E // _CHUNK, _CHUNK)
    ea2 = ea.reshape(_E // _CHUNK, _CHUNK)
    return f(t, src2, dst2, ea2)


_CBLK = 3200


def _combine_body(acc_ref, hroot_ref, out_ref):
    s = acc_ref[0, :, 0, :] + acc_ref[1, :, 0, :]
    c = acc_ref[0, :, 1, :] + acc_ref[1, :, 1, :]
    out_ref[...] = (
        jnp.where(c > 0.0, s / jnp.maximum(c, 1.0), 0.0) + hroot_ref[...])


def _combine(acc, hroot):
    acc4 = acc.reshape(_NC, _N, 2, _D)
    return pl.pallas_call(
        _combine_body,
        grid=(_N // _CBLK,),
        in_specs=[
            pl.BlockSpec((_NC, _CBLK, 2, _D), lambda ib: (0, ib, 0, 0)),
            pl.BlockSpec((_CBLK, _D), lambda ib: (ib, 0)),
        ],
        out_specs=pl.BlockSpec((_CBLK, _D), lambda ib: (ib, 0)),
        out_shape=jax.ShapeDtypeStruct((_N, _D), jnp.float32),
    )(acc4, hroot)


def kernel(x, edge_index, edge_attribute, i, dummy,
           W_u, b_u, W_l1, b_l1, root, bias):
    xf = x.reshape(_N, 1)
    src = edge_index[0]
    dst = edge_index[1]
    ea = edge_attribute.reshape(_E)
    wpq = jnp.concatenate(
        [W_l1.reshape(_D, _D), b_l1.reshape(_D, _D)], axis=1)
    t, hroot = _node_phase(xf, W_u, b_u.reshape(1, _D), wpq,
                           root, bias.reshape(1, _D))
    acc = _edge_phase(t, src, dst, ea)
    return _combine(acc, hroot)


# speedup vs baseline: 22.0771x; 1.6058x over previous
import functools

import jax
import jax.numpy as jnp
from jax import lax
from jax.experimental import pallas as pl
from jax.experimental.pallas import tpu as pltpu
from jax.experimental.pallas import tpu_sc as plsc

_B = 128
_U = 200
_D = 16
_N = _B * _U
_E = 409600
_NC = 2
_NS = 16
_TILE_EDGES = _E // (_NC * _NS)
_CHUNK = 128
_NCHUNK = _TILE_EDGES // _CHUNK
_ROWS_PER_TILE = _N // _NS
_ZROWS = 100


def _node_body(xf_ref, wu_ref, bu_ref, wpq_ref, root_ref, bias_ref,
               t_ref, hroot_ref):
    h = jnp.maximum(xf_ref[...] * wu_ref[...] + bu_ref[...], 0.0)
    t_ref[...] = jnp.dot(h, wpq_ref[...], preferred_element_type=jnp.float32)
    hroot_ref[...] = (
        jnp.dot(h, root_ref[...], preferred_element_type=jnp.float32)
        + bias_ref[...])


def _node_phase(xf, wu, bu, wpq, root, bias):
    return pl.pallas_call(
        _node_body,
        out_shape=(
            jax.ShapeDtypeStruct((_N, 2 * _D), jnp.float32),
            jax.ShapeDtypeStruct((_N, _D), jnp.float32),
        ),
    )(xf, wu, bu, wpq, root, bias)


def _sc_body(t_hbm, src_hbm, dst_hbm, ea_hbm, out_hbm,
             sall, dall, eall, rows, msg, zbuf, acc_sh, gsem):
    cid = lax.axis_index("c")
    sid = lax.axis_index("s")
    wid = cid * _NS + sid

    pltpu.sync_copy(src_hbm.at[pl.ds(wid * _NCHUNK, _NCHUNK)], sall)
    pltpu.sync_copy(dst_hbm.at[pl.ds(wid * _NCHUNK, _NCHUNK)], dall)
    pltpu.sync_copy(ea_hbm.at[pl.ds(wid * _NCHUNK, _NCHUNK)], eall)

    zero16 = jnp.zeros((_D,), jnp.float32)

    def zfill(j, carry):
        zbuf[j, pl.ds(0, _D)] = zero16
        zbuf[j, pl.ds(_D, _D)] = zero16
        return carry

    lax.fori_loop(0, _ZROWS, zfill, 0)
    row0 = sid * _ROWS_PER_TILE
    for k in range(_ROWS_PER_TILE // _ZROWS):
        pltpu.sync_copy(zbuf, acc_sh.at[pl.ds(row0 + k * _ZROWS, _ZROWS)])

    one16 = jnp.ones((_D,), jnp.float32)

    def ofill(j, carry):
        msg[0, j, pl.ds(_D, _D)] = one16
        msg[1, j, pl.ds(_D, _D)] = one16
        return carry

    lax.fori_loop(0, _CHUNK, ofill, 0)
    plsc.subcore_barrier()

    pltpu.async_copy(t_hbm.at[sall.at[0]], rows.at[0], gsem)

    def do_chunk(ci, b):
        nci = ci + 1

        @pl.when(nci < _NCHUNK)
        def _():
            pltpu.async_copy(t_hbm.at[sall.at[nci]], rows.at[1 - b], gsem)

        pltpu.make_async_copy(t_hbm.at[sall.at[ci]], rows.at[b], gsem).wait()

        def group_body(g, c2):
            base = g * _D
            ev = eall[ci, pl.ds(base, _D)]
            for k in range(_D):
                j = base + k
                p = rows[b, j, pl.ds(0, _D)]
                q = rows[b, j, pl.ds(_D, _D)]
                msg[b, j, pl.ds(0, _D)] = p * ev[k] + q
            return c2

        lax.fori_loop(0, _CHUNK // _D, group_body, 0)
        pltpu.sync_copy(msg.at[b], acc_sh.at[dall.at[ci]], add=True)

    def pair_body(h, carry):
        do_chunk(h * 2, 0)
        do_chunk(h * 2 + 1, 1)
        return carry

    lax.fori_loop(0, _NCHUNK // 2, pair_body, 0)
    plsc.subcore_barrier()

    pltpu.sync_copy(acc_sh.at[pl.ds(row0, _ROWS_PER_TILE)],
                    out_hbm.at[cid, pl.ds(row0, _ROWS_PER_TILE)])


def _edge_phase(t, src, dst, ea):
    mesh = plsc.VectorSubcoreMesh(core_axis_name="c", subcore_axis_name="s")
    f = pl.kernel(
        _sc_body,
        mesh=mesh,
        compiler_params=pltpu.CompilerParams(use_tc_tiling_on_sc=False),
        out_type=jax.ShapeDtypeStruct((_NC, _N, 2 * _D), jnp.float32),
        scratch_types=[
            pltpu.VMEM((_NCHUNK, _CHUNK), jnp.int32),
            pltpu.VMEM((_NCHUNK, _CHUNK), jnp.int32),
            pltpu.VMEM((_NCHUNK, _CHUNK), jnp.float32),
            pltpu.VMEM((2, _CHUNK, 2 * _D), jnp.float32),
            pltpu.VMEM((2, _CHUNK, 2 * _D), jnp.float32),
            pltpu.VMEM((_ZROWS, 2 * _D), jnp.float32),
            pltpu.VMEM_SHARED((_N, 2 * _D), jnp.float32),
            pltpu.SemaphoreType.DMA,
        ],
    )
    src2 = src.reshape(_E // _CHUNK, _CHUNK)
    dst2 = dst.reshape(_E // _CHUNK, _CHUNK)
    ea2 = ea.reshape(_E // _CHUNK, _CHUNK)
    return f(t, src2, dst2, ea2)


_CSTRIPE = _N // (_NC * _NS)


def _combine_body(acc_hbm, hroot_hbm, out_hbm, va, vb, vh, vo, _):
    cid = lax.axis_index("c")
    sid = lax.axis_index("s")
    wid = cid * _NS + sid
    n0 = wid * _CSTRIPE
    pltpu.sync_copy(acc_hbm.at[0, pl.ds(n0, _CSTRIPE)], va)
    pltpu.sync_copy(acc_hbm.at[1, pl.ds(n0, _CSTRIPE)], vb)
    pltpu.sync_copy(hroot_hbm.at[pl.ds(n0, _CSTRIPE)], vh)

    def node_group(g, carry):
        for k in range(_D):
            j = g * _D + k
            s = va[j, pl.ds(0, _D)] + vb[j, pl.ds(0, _D)]
            c = va[j, pl.ds(_D, _D)] + vb[j, pl.ds(_D, _D)]
            aggr = jnp.where(c > 0.0, s / jnp.maximum(c, 1.0), 0.0)
            vo[j, :] = aggr + vh[j, :]
        return carry

    lax.fori_loop(0, _CSTRIPE // _D, node_group, 0)
    pltpu.sync_copy(vo, out_hbm.at[pl.ds(n0, _CSTRIPE)])


def _combine(acc, hroot):
    mesh = plsc.VectorSubcoreMesh(core_axis_name="c", subcore_axis_name="s")
    f = pl.kernel(
        _combine_body,
        mesh=mesh,
        compiler_params=pltpu.CompilerParams(use_tc_tiling_on_sc=False),
        out_type=jax.ShapeDtypeStruct((_N, _D), jnp.float32),
        scratch_types=[
            pltpu.VMEM((_CSTRIPE, 2 * _D), jnp.float32),
            pltpu.VMEM((_CSTRIPE, 2 * _D), jnp.float32),
            pltpu.VMEM((_CSTRIPE, _D), jnp.float32),
            pltpu.VMEM((_CSTRIPE, _D), jnp.float32),
            pltpu.SemaphoreType.DMA,
        ],
    )
    return f(acc, hroot)


def kernel(x, edge_index, edge_attribute, i, dummy,
           W_u, b_u, W_l1, b_l1, root, bias):
    xf = x.reshape(_N, 1)
    src = edge_index[0]
    dst = edge_index[1]
    ea = edge_attribute.reshape(_E)
    wpq = jnp.concatenate(
        [W_l1.reshape(_D, _D), b_l1.reshape(_D, _D)], axis=1)
    t, hroot = _node_phase(xf, W_u, b_u.reshape(1, _D), wpq,
                           root, bias.reshape(1, _D))
    acc = _edge_phase(t, src, dst, ea)
    return _combine(acc, hroot)
